# baseline (speedup 1.0000x reference)
import jax
import jax.numpy as jnp
from jax import lax
from jax.experimental import pallas as pl
from jax.experimental.pallas import tpu as pltpu

D_MODEL = 2048
N_HEADS = 16
N_KV_GROUPS = 4
HPG = N_HEADS // N_KV_GROUPS
D_QK = 128
D_V = 128
BLK = 64
NB = 2048 // BLK
TOP_N = 16
WINDOW = 512
SCALE = 1.0 / (D_QK ** 0.5)
S = 2048
QT = 256
KT = 256

_f32 = jnp.float32
_bf16 = jnp.bfloat16


def _bf(a):
    return a.astype(_bf16)


def _proj_bf_kernel(x_ref, w_ref, y_ref):
    y_ref[...] = _bf(jnp.dot(x_ref[...], w_ref[...],
                             preferred_element_type=_f32))


def _proj_f32_kernel(x_ref, w_ref, y_ref):
    y_ref[...] = jnp.dot(x_ref[...], w_ref[...], preferred_element_type=_f32)


def _project(x2, w, out_dtype, bn):
    N = w.shape[1]
    bm = 512
    kern = _proj_bf_kernel if out_dtype == _bf16 else _proj_f32_kernel
    return pl.pallas_call(
        kern,
        grid=(S // bm, N // bn),
        compiler_params=pltpu.CompilerParams(
            dimension_semantics=("parallel", "parallel")),
        in_specs=[
            pl.BlockSpec((bm, D_MODEL), lambda i, j: (i, 0)),
            pl.BlockSpec((D_MODEL, bn), lambda i, j: (0, j)),
        ],
        out_specs=pl.BlockSpec((bm, bn), lambda i, j: (i, j)),
        out_shape=jax.ShapeDtypeStruct((S, N), out_dtype),
    )(x2, w)


def _pool_kernel(k_ref, v_ref, kc_ref, vc_ref):
    kc_ref[0] = jnp.mean(k_ref[...].reshape(NB, BLK, D_QK), axis=1)
    vc_ref[0] = jnp.mean(v_ref[...].reshape(NB, BLK, D_V), axis=1)


def _pool(k_f32, v_f32):
    return pl.pallas_call(
        _pool_kernel,
        grid=(N_KV_GROUPS,),
        compiler_params=pltpu.CompilerParams(
            dimension_semantics=("parallel",)),
        in_specs=[
            pl.BlockSpec((S, D_QK), lambda g: (0, g)),
            pl.BlockSpec((S, D_V), lambda g: (0, g)),
        ],
        out_specs=[
            pl.BlockSpec((1, NB, D_QK), lambda g: (g, 0, 0)),
            pl.BlockSpec((1, NB, D_V), lambda g: (g, 0, 0)),
        ],
        out_shape=[
            jax.ShapeDtypeStruct((N_KV_GROUPS, NB, D_QK), _f32),
            jax.ShapeDtypeStruct((N_KV_GROUPS, NB, D_V), _f32),
        ],
    )(k_f32, v_f32)


def _upd(mla, s_masked, v_b):
    m0, l0, a0 = mla
    mn = jnp.maximum(m0, jnp.max(s_masked, axis=1, keepdims=True))
    p = jnp.exp(s_masked - mn)
    alpha = jnp.exp(m0 - mn)
    l1 = l0 * alpha + jnp.sum(p, axis=1, keepdims=True)
    a1 = a0 * alpha + jnp.dot(_bf(p), v_b, preferred_element_type=_f32)
    return (mn, l1, a1)


def _attn_kernel(q_ref, kt_ref, v_ref, kc_ref, vc_ref, exp_ref, g_ref,
                 o_ref, selm_ref):
    qt = pl.program_id(1)
    q = q_ref[...]
    kc_b = _bf(kc_ref[0])
    vc_b = _bf(vc_ref[0])
    posn = lax.broadcasted_iota(jnp.int32, (QT, NB), 0) + qt * QT
    blkn = lax.broadcasted_iota(jnp.int32, (QT, NB), 1)
    cmask = ((blkn + 1) * BLK - 1 <= posn).astype(_f32)
    imp = jnp.zeros((QT, NB), _f32)
    oc = []
    for h in range(HPG):
        s = lax.dot_general(q[:, h * D_QK:(h + 1) * D_QK], kc_b,
                            (((1,), (1,)), ((), ())),
                            preferred_element_type=_f32) * SCALE
        s = jnp.where(cmask > 0, s, -1e30)
        m = jnp.max(s, axis=1, keepdims=True)
        m = jnp.where(m > -1e29, m, 0.0)
        p = jnp.exp(s - m) * cmask
        den = jnp.sum(p, axis=1, keepdims=True)
        pn = p / jnp.maximum(den, 1e-9)
        oc.append(jnp.dot(_bf(pn), vc_b, preferred_element_type=_f32))
        imp = imp + pn * (1.0 / HPG)
    cols = []
    for n in range(NB):
        cn = imp[:, n:n + 1]
        beats = (imp > cn).astype(jnp.int32)
        if n > 0:
            beats = beats + jnp.where(blkn < n, (imp == cn).astype(jnp.int32), 0)
        cols.append(jnp.sum(beats, axis=1, keepdims=True))
    rank = jnp.concatenate(cols, axis=1)
    own = (posn // BLK) == blkn
    sel = ((rank < TOP_N) | own).astype(_f32)
    selm_ref[...] = jnp.dot(_bf(sel), exp_ref[...],
                            preferred_element_type=_f32)
    iq = lax.broadcasted_iota(jnp.int32, (QT, KT), 0)
    ic = lax.broadcasted_iota(jnp.int32, (QT, KT), 1)
    rel = iq - ic
    causal = rel >= 0
    neg_inf = jnp.float32(-jnp.inf)

    def zero3():
        return (jnp.full((QT, 1), -1e30, _f32), jnp.zeros((QT, 1), _f32),
                jnp.zeros((QT, D_V), _f32))

    carry = tuple(x for _ in range(2 * HPG) for x in zero3())

    def tile_body(kb, c, with_win):
        c = list(c)
        k_t = kt_ref[0, :, pl.ds(kb * KT, KT)]
        v_t = v_ref[0, pl.ds(kb * KT, KT), :]
        selb = selm_ref[:, pl.ds(kb * KT, KT)] > 0
        win_cut = WINDOW - (qt - kb) * KT
        for h in range(HPG):
            s = jnp.dot(q[:, h * D_QK:(h + 1) * D_QK], k_t,
                        preferred_element_type=_f32) * SCALE
            c[3 * h:3 * h + 3] = _upd(tuple(c[3 * h:3 * h + 3]),
                                      jnp.where(selb, s, neg_inf), v_t)
            if with_win:
                j = 3 * (HPG + h)
                c[j:j + 3] = _upd(tuple(c[j:j + 3]),
                                  jnp.where(rel < win_cut, s, neg_inf), v_t)
        return tuple(c)

    t0 = jnp.maximum(qt - WINDOW // KT, 0)
    carry = lax.fori_loop(0, t0, lambda kb, c: tile_body(kb, c, False), carry)
    carry = lax.fori_loop(t0, qt, lambda kb, c: tile_body(kb, c, True), carry)
    c = list(carry)
    k_t = kt_ref[0, :, pl.ds(qt * KT, KT)]
    v_t = v_ref[0, pl.ds(qt * KT, KT), :]
    selb = (selm_ref[:, pl.ds(qt * KT, KT)] > 0) & causal
    for h in range(HPG):
        s = jnp.dot(q[:, h * D_QK:(h + 1) * D_QK], k_t,
                    preferred_element_type=_f32) * SCALE
        c[3 * h:3 * h + 3] = _upd(tuple(c[3 * h:3 * h + 3]),
                                  jnp.where(selb, s, neg_inf), v_t)
        j = 3 * (HPG + h)
        c[j:j + 3] = _upd(tuple(c[j:j + 3]), jnp.where(causal, s, neg_inf),
                          v_t)
    g = jax.nn.sigmoid(g_ref[0])
    for h in range(HPG):
        _, l_s, a_s = c[3 * h:3 * h + 3]
        _, l_w, a_w = c[3 * (HPG + h):3 * (HPG + h) + 3]
        o_sel = a_s / jnp.maximum(l_s, 1e-9)
        o_win = a_w / jnp.maximum(l_w, 1e-9)
        o = (g[:, 3 * h:3 * h + 1] * oc[h]
             + g[:, 3 * h + 1:3 * h + 2] * o_sel
             + g[:, 3 * h + 2:3 * h + 3] * o_win)
        o_ref[:, h * D_V:(h + 1) * D_V] = _bf(o)


def _attention(q_bf, kt_bf, v_bf, kc, vc, expand_bf, glog):
    return pl.pallas_call(
        _attn_kernel,
        grid=(N_KV_GROUPS, S // QT),
        compiler_params=pltpu.CompilerParams(
            dimension_semantics=("parallel", "parallel")),
        in_specs=[
            pl.BlockSpec((QT, HPG * D_QK), lambda g, t: (t, g)),
            pl.BlockSpec((1, D_QK, S), lambda g, t: (g, 0, 0)),
            pl.BlockSpec((1, S, D_V), lambda g, t: (g, 0, 0)),
            pl.BlockSpec((1, NB, D_QK), lambda g, t: (g, 0, 0)),
            pl.BlockSpec((1, NB, D_V), lambda g, t: (g, 0, 0)),
            pl.BlockSpec((NB, S), lambda g, t: (0, 0)),
            pl.BlockSpec((1, QT, 3 * HPG), lambda g, t: (g, t, 0)),
        ],
        out_specs=pl.BlockSpec((QT, HPG * D_V), lambda g, t: (t, g)),
        out_shape=jax.ShapeDtypeStruct((S, N_HEADS * D_V), _bf16),
        scratch_shapes=[pltpu.VMEM((QT, S), _f32)],
    )(q_bf, kt_bf, v_bf, kc, vc, expand_bf, glog)


def _out_kernel(o_ref, w_ref, y_ref):
    y_ref[...] = jnp.dot(o_ref[...], w_ref[...], preferred_element_type=_f32)


def _out_proj(o_bf, wo_bf):
    bm, bn = 512, 512
    return pl.pallas_call(
        _out_kernel,
        grid=(S // bm, D_MODEL // bn),
        compiler_params=pltpu.CompilerParams(
            dimension_semantics=("parallel", "parallel")),
        in_specs=[
            pl.BlockSpec((bm, N_HEADS * D_V), lambda i, j: (i, 0)),
            pl.BlockSpec((N_HEADS * D_V, bn), lambda i, j: (0, j)),
        ],
        out_specs=pl.BlockSpec((bm, bn), lambda i, j: (i, j)),
        out_shape=jax.ShapeDtypeStruct((S, D_MODEL), _f32),
    )(o_bf, wo_bf)


def kernel(x, Wq, Wk, Wv, Wg, Wo):
    x2_bf = _bf(x[0])
    q_bf = _project(x2_bf, _bf(Wq), _bf16, 512)
    w_kvg = jnp.concatenate(
        [Wk, Wv, Wg, jnp.zeros((D_MODEL, 208), _f32)], axis=1)
    y2 = _project(x2_bf, _bf(w_kvg), _f32, 640)
    k = y2[:, :512]
    v = y2[:, 512:1024]
    glog = y2[:, 1024:1072].reshape(S, N_KV_GROUPS, 3 * HPG).transpose(1, 0, 2)
    kc, vc = _pool(k, v)
    kt_bf = _bf(k).reshape(S, N_KV_GROUPS, D_QK).transpose(1, 2, 0)
    v_bf = _bf(v).reshape(S, N_KV_GROUPS, D_V).transpose(1, 0, 2)
    expand_bf = (jnp.arange(S)[None, :] // BLK
                 == jnp.arange(NB)[:, None]).astype(_bf16)
    o_bf = _attention(q_bf, kt_bf, v_bf, kc, vc, expand_bf, glog)
    out = _out_proj(o_bf, _bf(Wo))
    return out[None]


# speedup vs baseline: 1.0867x; 1.0867x over previous
import jax
import jax.numpy as jnp
from jax import lax
from jax.experimental import pallas as pl
from jax.experimental.pallas import tpu as pltpu

D_MODEL = 2048
N_HEADS = 16
N_KV_GROUPS = 4
HPG = N_HEADS // N_KV_GROUPS
D_QK = 128
D_V = 128
BLK = 64
NB = 2048 // BLK
TOP_N = 16
WINDOW = 512
SCALE = 1.0 / (D_QK ** 0.5)
S = 2048
QT = 256
KT = 256

_f32 = jnp.float32
_bf16 = jnp.bfloat16


def _bf(a):
    return a.astype(_bf16)


def _proj_bf_kernel(x_ref, w_ref, y_ref):
    y_ref[...] = _bf(jnp.dot(x_ref[...], w_ref[...],
                             preferred_element_type=_f32))


def _proj_f32_kernel(x_ref, w_ref, y_ref):
    y_ref[...] = jnp.dot(x_ref[...], w_ref[...], preferred_element_type=_f32)


def _project(x2, w, out_dtype, bn):
    N = w.shape[1]
    bm = 512
    kern = _proj_bf_kernel if out_dtype == _bf16 else _proj_f32_kernel
    return pl.pallas_call(
        kern,
        grid=(S // bm, N // bn),
        compiler_params=pltpu.CompilerParams(
            dimension_semantics=("parallel", "parallel")),
        in_specs=[
            pl.BlockSpec((bm, D_MODEL), lambda i, j: (i, 0)),
            pl.BlockSpec((D_MODEL, bn), lambda i, j: (0, j)),
        ],
        out_specs=pl.BlockSpec((bm, bn), lambda i, j: (i, j)),
        out_shape=jax.ShapeDtypeStruct((S, N), out_dtype),
    )(x2, w)


def _cmp_kernel(q_ref, k_ref, v_ref, oc_ref, sel_ref):
    kc_b = _bf(jnp.mean(k_ref[...].reshape(NB, BLK, D_QK), axis=1))
    vc_b = _bf(jnp.mean(v_ref[...].reshape(NB, BLK, D_V), axis=1))
    pos = lax.broadcasted_iota(jnp.int32, (S, NB), 0)
    blk = lax.broadcasted_iota(jnp.int32, (S, NB), 1)
    cmask = ((blk + 1) * BLK - 1 <= pos).astype(_f32)
    imp = jnp.zeros((S, NB), _f32)
    for h in range(HPG):
        qh = q_ref[:, h * D_QK:(h + 1) * D_QK]
        s = lax.dot_general(qh, kc_b, (((1,), (1,)), ((), ())),
                            preferred_element_type=_f32) * SCALE
        s = jnp.where(cmask > 0, s, -1e30)
        m = jnp.max(s, axis=1, keepdims=True)
        m = jnp.where(m > -1e29, m, 0.0)
        p = jnp.exp(s - m) * cmask
        den = jnp.sum(p, axis=1, keepdims=True)
        pn = p / jnp.maximum(den, 1e-9)
        oc_ref[:, h * D_V:(h + 1) * D_V] = jnp.dot(
            _bf(pn), vc_b, preferred_element_type=_f32)
        imp = imp + pn * (1.0 / HPG)
    cols = []
    for n in range(NB):
        cn = imp[:, n:n + 1]
        beats = (imp > cn).astype(jnp.int32)
        if n > 0:
            beats = beats + jnp.where(blk < n, (imp == cn).astype(jnp.int32), 0)
        cols.append(jnp.sum(beats, axis=1, keepdims=True))
    rank = jnp.concatenate(cols, axis=1)
    own = (pos // BLK) == blk
    sel = (rank < TOP_N) | own
    sel_ref[0] = sel.astype(_f32)


def _compress_select(q_bf, k_f32, v_f32):
    return pl.pallas_call(
        _cmp_kernel,
        grid=(N_KV_GROUPS,),
        compiler_params=pltpu.CompilerParams(
            dimension_semantics=("parallel",)),
        in_specs=[
            pl.BlockSpec((S, HPG * D_QK), lambda g: (0, g)),
            pl.BlockSpec((S, D_QK), lambda g: (0, g)),
            pl.BlockSpec((S, D_V), lambda g: (0, g)),
        ],
        out_specs=[
            pl.BlockSpec((S, HPG * D_V), lambda g: (0, g)),
            pl.BlockSpec((1, S, NB), lambda g: (g, 0, 0)),
        ],
        out_shape=[
            jax.ShapeDtypeStruct((S, N_HEADS * D_V), _f32),
            jax.ShapeDtypeStruct((N_KV_GROUPS, S, NB), _f32),
        ],
    )(q_bf, k_f32, v_f32)


def _upd(mla, s_masked, v_b):
    m0, l0, a0 = mla
    mn = jnp.maximum(m0, jnp.max(s_masked, axis=1, keepdims=True))
    p = jnp.exp(s_masked - mn)
    alpha = jnp.exp(m0 - mn)
    l1 = l0 * alpha + jnp.sum(p, axis=1, keepdims=True)
    a1 = a0 * alpha + jnp.dot(_bf(p), v_b, preferred_element_type=_f32)
    return (mn, l1, a1)


def _attn_kernel(q_ref, kt_ref, v_ref, sel_ref, exp_ref, oc_ref, g_ref,
                 o_ref, selm_ref):
    qt = pl.program_id(1)
    q = q_ref[...]
    selm_ref[...] = jnp.dot(_bf(sel_ref[0]), exp_ref[...],
                            preferred_element_type=_f32)
    iq = lax.broadcasted_iota(jnp.int32, (QT, KT), 0)
    ic = lax.broadcasted_iota(jnp.int32, (QT, KT), 1)
    rel = iq - ic
    causal = rel >= 0
    neg_inf = jnp.float32(-jnp.inf)

    def zero3():
        return (jnp.full((QT, 1), -1e30, _f32), jnp.zeros((QT, 1), _f32),
                jnp.zeros((QT, D_V), _f32))

    carry = tuple(x for _ in range(2 * HPG) for x in zero3())

    def tile_body(kb, c, with_win):
        c = list(c)
        k_t = kt_ref[0, :, pl.ds(kb * KT, KT)]
        v_t = v_ref[0, pl.ds(kb * KT, KT), :]
        selb = selm_ref[:, pl.ds(kb * KT, KT)] > 0
        win_cut = WINDOW - (qt - kb) * KT
        for h in range(HPG):
            s = jnp.dot(q[:, h * D_QK:(h + 1) * D_QK], k_t,
                        preferred_element_type=_f32) * SCALE
            c[3 * h:3 * h + 3] = _upd(tuple(c[3 * h:3 * h + 3]),
                                      jnp.where(selb, s, neg_inf), v_t)
            if with_win:
                j = 3 * (HPG + h)
                c[j:j + 3] = _upd(tuple(c[j:j + 3]),
                                  jnp.where(rel < win_cut, s, neg_inf), v_t)
        return tuple(c)

    t0 = jnp.maximum(qt - WINDOW // KT, 0)
    carry = lax.fori_loop(0, t0, lambda kb, c: tile_body(kb, c, False), carry)
    carry = lax.fori_loop(t0, qt, lambda kb, c: tile_body(kb, c, True), carry)
    c = list(carry)
    k_t = kt_ref[0, :, pl.ds(qt * KT, KT)]
    v_t = v_ref[0, pl.ds(qt * KT, KT), :]
    selb = (selm_ref[:, pl.ds(qt * KT, KT)] > 0) & causal
    for h in range(HPG):
        s = jnp.dot(q[:, h * D_QK:(h + 1) * D_QK], k_t,
                    preferred_element_type=_f32) * SCALE
        c[3 * h:3 * h + 3] = _upd(tuple(c[3 * h:3 * h + 3]),
                                  jnp.where(selb, s, neg_inf), v_t)
        j = 3 * (HPG + h)
        c[j:j + 3] = _upd(tuple(c[j:j + 3]), jnp.where(causal, s, neg_inf),
                          v_t)
    g = jax.nn.sigmoid(g_ref[0])
    for h in range(HPG):
        _, l_s, a_s = c[3 * h:3 * h + 3]
        _, l_w, a_w = c[3 * (HPG + h):3 * (HPG + h) + 3]
        o_sel = a_s / jnp.maximum(l_s, 1e-9)
        o_win = a_w / jnp.maximum(l_w, 1e-9)
        o = (g[:, 3 * h:3 * h + 1] * oc_ref[:, h * D_V:(h + 1) * D_V]
             + g[:, 3 * h + 1:3 * h + 2] * o_sel
             + g[:, 3 * h + 2:3 * h + 3] * o_win)
        o_ref[:, h * D_V:(h + 1) * D_V] = _bf(o)


def _attention(q_bf, kt_bf, v_bf, sel, expand_bf, out_cmp, glog):
    return pl.pallas_call(
        _attn_kernel,
        grid=(N_KV_GROUPS, S // QT),
        compiler_params=pltpu.CompilerParams(
            dimension_semantics=("parallel", "parallel")),
        in_specs=[
            pl.BlockSpec((QT, HPG * D_QK), lambda g, t: (t, g)),
            pl.BlockSpec((1, D_QK, S), lambda g, t: (g, 0, 0)),
            pl.BlockSpec((1, S, D_V), lambda g, t: (g, 0, 0)),
            pl.BlockSpec((1, QT, NB), lambda g, t: (g, t, 0)),
            pl.BlockSpec((NB, S), lambda g, t: (0, 0)),
            pl.BlockSpec((QT, HPG * D_V), lambda g, t: (t, g)),
            pl.BlockSpec((1, QT, 3 * HPG), lambda g, t: (g, t, 0)),
        ],
        out_specs=pl.BlockSpec((QT, HPG * D_V), lambda g, t: (t, g)),
        out_shape=jax.ShapeDtypeStruct((S, N_HEADS * D_V), _bf16),
        scratch_shapes=[pltpu.VMEM((QT, S), _f32)],
    )(q_bf, kt_bf, v_bf, sel, expand_bf, out_cmp, glog)


def _out_kernel(o_ref, w_ref, y_ref):
    y_ref[...] = jnp.dot(o_ref[...], w_ref[...], preferred_element_type=_f32)


def _out_proj(o_bf, wo_bf):
    bm, bn = 512, 512
    return pl.pallas_call(
        _out_kernel,
        grid=(S // bm, D_MODEL // bn),
        compiler_params=pltpu.CompilerParams(
            dimension_semantics=("parallel", "parallel")),
        in_specs=[
            pl.BlockSpec((bm, N_HEADS * D_V), lambda i, j: (i, 0)),
            pl.BlockSpec((N_HEADS * D_V, bn), lambda i, j: (0, j)),
        ],
        out_specs=pl.BlockSpec((bm, bn), lambda i, j: (i, j)),
        out_shape=jax.ShapeDtypeStruct((S, D_MODEL), _f32),
    )(o_bf, wo_bf)


def kernel(x, Wq, Wk, Wv, Wg, Wo):
    x2_bf = _bf(x[0])
    q_bf = _project(x2_bf, _bf(Wq), _bf16, 512)
    w_kvg = jnp.concatenate(
        [Wk, Wv, Wg, jnp.zeros((D_MODEL, 208), _f32)], axis=1)
    y2 = _project(x2_bf, _bf(w_kvg), _f32, 640)
    k = y2[:, :512]
    v = y2[:, 512:1024]
    glog = y2[:, 1024:1072].reshape(S, N_KV_GROUPS, 3 * HPG).transpose(1, 0, 2)
    out_cmp, sel = _compress_select(q_bf, k, v)
    kt_bf = _bf(k).reshape(S, N_KV_GROUPS, D_QK).transpose(1, 2, 0)
    v_bf = _bf(v).reshape(S, N_KV_GROUPS, D_V).transpose(1, 0, 2)
    expand_bf = (jnp.arange(S)[None, :] // BLK
                 == jnp.arange(NB)[:, None]).astype(_bf16)
    o_bf = _attention(q_bf, kt_bf, v_bf, sel, expand_bf, out_cmp, glog)
    out = _out_proj(o_bf, _bf(Wo))
    return out[None]


# speedup vs baseline: 1.4262x; 1.3124x over previous
import jax
import jax.numpy as jnp
from jax import lax
from jax.experimental import pallas as pl
from jax.experimental.pallas import tpu as pltpu

D_MODEL = 2048
N_HEADS = 16
N_KV_GROUPS = 4
HPG = N_HEADS // N_KV_GROUPS
D_QK = 128
D_V = 128
BLK = 64
NB = 2048 // BLK
TOP_N = 16
WINDOW = 512
SCALE = 1.0 / (D_QK ** 0.5)
S = 2048
QT = 256
KT = 256

_f32 = jnp.float32
_bf16 = jnp.bfloat16


def _bf(a):
    return a.astype(_bf16)


def _proj_bf_kernel(x_ref, w_ref, y_ref):
    y_ref[...] = _bf(jnp.dot(x_ref[...], w_ref[...],
                             preferred_element_type=_f32))


def _proj_f32_kernel(x_ref, w_ref, y_ref):
    y_ref[...] = jnp.dot(x_ref[...], w_ref[...], preferred_element_type=_f32)


def _project(x2, w, out_dtype, bn):
    N = w.shape[1]
    bm = 512
    kern = _proj_bf_kernel if out_dtype == _bf16 else _proj_f32_kernel
    return pl.pallas_call(
        kern,
        grid=(S // bm, N // bn),
        compiler_params=pltpu.CompilerParams(
            dimension_semantics=("parallel", "parallel")),
        in_specs=[
            pl.BlockSpec((bm, D_MODEL), lambda i, j: (i, 0)),
            pl.BlockSpec((D_MODEL, bn), lambda i, j: (0, j)),
        ],
        out_specs=pl.BlockSpec((bm, bn), lambda i, j: (i, j)),
        out_shape=jax.ShapeDtypeStruct((S, N), out_dtype),
    )(x2, w)


def _cmp_kernel(q_ref, k_ref, v_ref, oc_ref, sel_ref):
    kc_b = _bf(jnp.mean(k_ref[...].reshape(NB, BLK, D_QK), axis=1))
    vc_b = _bf(jnp.mean(v_ref[...].reshape(NB, BLK, D_V), axis=1))
    pos = lax.broadcasted_iota(jnp.int32, (NB, S), 1)
    blk = lax.broadcasted_iota(jnp.int32, (NB, S), 0)
    cmask = ((blk + 1) * BLK - 1 <= pos).astype(_f32)
    imp = jnp.zeros((NB, S), _f32)
    for h in range(HPG):
        qh = q_ref[:, h * D_QK:(h + 1) * D_QK]
        s = lax.dot_general(kc_b, qh, (((1,), (1,)), ((), ())),
                            preferred_element_type=_f32) * SCALE
        s = jnp.where(cmask > 0, s, -1e30)
        m = jnp.max(s, axis=0, keepdims=True)
        m = jnp.where(m > -1e29, m, 0.0)
        p = jnp.exp(s - m) * cmask
        den = jnp.sum(p, axis=0, keepdims=True)
        pn = p / jnp.maximum(den, 1e-9)
        oc_ref[:, h * D_V:(h + 1) * D_V] = lax.dot_general(
            _bf(pn), vc_b, (((0,), (0,)), ((), ())),
            preferred_element_type=_f32)
        imp = imp + pn * (1.0 / HPG)
    rows = []
    for n in range(NB):
        cn = imp[n:n + 1, :]
        beats = (imp > cn).astype(jnp.int32)
        if n > 0:
            beats = beats + jnp.where(blk < n, (imp == cn).astype(jnp.int32), 0)
        rows.append(jnp.sum(beats, axis=0, keepdims=True))
    rank = jnp.concatenate(rows, axis=0)
    own = (pos // BLK) == blk
    sel = (rank < TOP_N) | own
    sel_ref[0] = sel.astype(_f32)


def _compress_select(q_bf, k_f32, v_f32):
    return pl.pallas_call(
        _cmp_kernel,
        grid=(N_KV_GROUPS,),
        compiler_params=pltpu.CompilerParams(
            dimension_semantics=("parallel",)),
        in_specs=[
            pl.BlockSpec((S, HPG * D_QK), lambda g: (0, g)),
            pl.BlockSpec((S, D_QK), lambda g: (0, g)),
            pl.BlockSpec((S, D_V), lambda g: (0, g)),
        ],
        out_specs=[
            pl.BlockSpec((S, HPG * D_V), lambda g: (0, g)),
            pl.BlockSpec((1, NB, S), lambda g: (g, 0, 0)),
        ],
        out_shape=[
            jax.ShapeDtypeStruct((S, N_HEADS * D_V), _f32),
            jax.ShapeDtypeStruct((N_KV_GROUPS, NB, S), _f32),
        ],
    )(q_bf, k_f32, v_f32)


def _upd(mla, s_masked, v_b):
    m0, l0, a0 = mla
    mn = jnp.maximum(m0, jnp.max(s_masked, axis=1, keepdims=True))
    p = jnp.exp(s_masked - mn)
    alpha = jnp.exp(m0 - mn)
    l1 = l0 * alpha + jnp.sum(p, axis=1, keepdims=True)
    a1 = a0 * alpha + jnp.dot(_bf(p), v_b, preferred_element_type=_f32)
    return (mn, l1, a1)


def _attn_kernel(q_ref, kt_ref, v_ref, sel_ref, exp_ref, oc_ref, g_ref,
                 o_ref, selm_ref):
    qt = pl.program_id(1)
    q = q_ref[...]
    selm_ref[...] = lax.dot_general(_bf(sel_ref[0]), exp_ref[...],
                                    (((0,), (0,)), ((), ())),
                                    preferred_element_type=_f32)
    iq = lax.broadcasted_iota(jnp.int32, (QT, KT), 0)
    ic = lax.broadcasted_iota(jnp.int32, (QT, KT), 1)
    rel = iq - ic
    causal = rel >= 0
    neg_inf = jnp.float32(-jnp.inf)

    def zero3():
        return (jnp.full((QT, 1), -1e30, _f32), jnp.zeros((QT, 1), _f32),
                jnp.zeros((QT, D_V), _f32))

    carry = tuple(x for _ in range(2 * HPG) for x in zero3())

    def tile_body(kb, c, with_win):
        c = list(c)
        k_t = kt_ref[0, :, pl.ds(kb * KT, KT)]
        v_t = v_ref[0, pl.ds(kb * KT, KT), :]
        selb = selm_ref[:, pl.ds(kb * KT, KT)] > 0
        win_cut = WINDOW - (qt - kb) * KT
        for h in range(HPG):
            s = jnp.dot(q[:, h * D_QK:(h + 1) * D_QK], k_t,
                        preferred_element_type=_f32) * SCALE
            c[3 * h:3 * h + 3] = _upd(tuple(c[3 * h:3 * h + 3]),
                                      jnp.where(selb, s, neg_inf), v_t)
            if with_win:
                j = 3 * (HPG + h)
                c[j:j + 3] = _upd(tuple(c[j:j + 3]),
                                  jnp.where(rel < win_cut, s, neg_inf), v_t)
        return tuple(c)

    t0 = jnp.maximum(qt - WINDOW // KT, 0)
    carry = lax.fori_loop(0, t0, lambda kb, c: tile_body(kb, c, False), carry)
    carry = lax.fori_loop(t0, qt, lambda kb, c: tile_body(kb, c, True), carry)
    c = list(carry)
    k_t = kt_ref[0, :, pl.ds(qt * KT, KT)]
    v_t = v_ref[0, pl.ds(qt * KT, KT), :]
    selb = (selm_ref[:, pl.ds(qt * KT, KT)] > 0) & causal
    for h in range(HPG):
        s = jnp.dot(q[:, h * D_QK:(h + 1) * D_QK], k_t,
                    preferred_element_type=_f32) * SCALE
        c[3 * h:3 * h + 3] = _upd(tuple(c[3 * h:3 * h + 3]),
                                  jnp.where(selb, s, neg_inf), v_t)
        j = 3 * (HPG + h)
        c[j:j + 3] = _upd(tuple(c[j:j + 3]), jnp.where(causal, s, neg_inf),
                          v_t)
    g = jax.nn.sigmoid(g_ref[0])
    for h in range(HPG):
        _, l_s, a_s = c[3 * h:3 * h + 3]
        _, l_w, a_w = c[3 * (HPG + h):3 * (HPG + h) + 3]
        o_sel = a_s / jnp.maximum(l_s, 1e-9)
        o_win = a_w / jnp.maximum(l_w, 1e-9)
        o = (g[:, 3 * h:3 * h + 1] * oc_ref[:, h * D_V:(h + 1) * D_V]
             + g[:, 3 * h + 1:3 * h + 2] * o_sel
             + g[:, 3 * h + 2:3 * h + 3] * o_win)
        o_ref[:, h * D_V:(h + 1) * D_V] = _bf(o)


def _attention(q_bf, kt_bf, v_bf, sel, expand_bf, out_cmp, glog):
    return pl.pallas_call(
        _attn_kernel,
        grid=(N_KV_GROUPS, S // QT),
        compiler_params=pltpu.CompilerParams(
            dimension_semantics=("parallel", "parallel")),
        in_specs=[
            pl.BlockSpec((QT, HPG * D_QK), lambda g, t: (t, g)),
            pl.BlockSpec((1, D_QK, S), lambda g, t: (g, 0, 0)),
            pl.BlockSpec((1, S, D_V), lambda g, t: (g, 0, 0)),
            pl.BlockSpec((1, NB, QT), lambda g, t: (g, 0, t)),
            pl.BlockSpec((NB, S), lambda g, t: (0, 0)),
            pl.BlockSpec((QT, HPG * D_V), lambda g, t: (t, g)),
            pl.BlockSpec((1, QT, 3 * HPG), lambda g, t: (g, t, 0)),
        ],
        out_specs=pl.BlockSpec((QT, HPG * D_V), lambda g, t: (t, g)),
        out_shape=jax.ShapeDtypeStruct((S, N_HEADS * D_V), _bf16),
        scratch_shapes=[pltpu.VMEM((QT, S), _f32)],
    )(q_bf, kt_bf, v_bf, sel, expand_bf, out_cmp, glog)


def _out_kernel(o_ref, w_ref, y_ref):
    y_ref[...] = jnp.dot(o_ref[...], w_ref[...], preferred_element_type=_f32)


def _out_proj(o_bf, wo_bf):
    bm, bn = 512, 512
    return pl.pallas_call(
        _out_kernel,
        grid=(S // bm, D_MODEL // bn),
        compiler_params=pltpu.CompilerParams(
            dimension_semantics=("parallel", "parallel")),
        in_specs=[
            pl.BlockSpec((bm, N_HEADS * D_V), lambda i, j: (i, 0)),
            pl.BlockSpec((N_HEADS * D_V, bn), lambda i, j: (0, j)),
        ],
        out_specs=pl.BlockSpec((bm, bn), lambda i, j: (i, j)),
        out_shape=jax.ShapeDtypeStruct((S, D_MODEL), _f32),
    )(o_bf, wo_bf)


def kernel(x, Wq, Wk, Wv, Wg, Wo):
    x2_bf = _bf(x[0])
    q_bf = _project(x2_bf, _bf(Wq), _bf16, 512)
    w_kvg = jnp.concatenate(
        [Wk, Wv, Wg, jnp.zeros((D_MODEL, 208), _f32)], axis=1)
    y2 = _project(x2_bf, _bf(w_kvg), _f32, 640)
    k = y2[:, :512]
    v = y2[:, 512:1024]
    glog = y2[:, 1024:1072].reshape(S, N_KV_GROUPS, 3 * HPG).transpose(1, 0, 2)
    out_cmp, sel = _compress_select(q_bf, k, v)
    kt_bf = _bf(k).reshape(S, N_KV_GROUPS, D_QK).transpose(1, 2, 0)
    v_bf = _bf(v).reshape(S, N_KV_GROUPS, D_V).transpose(1, 0, 2)
    expand_bf = (jnp.arange(S)[None, :] // BLK
                 == jnp.arange(NB)[:, None]).astype(_bf16)
    o_bf = _attention(q_bf, kt_bf, v_bf, sel, expand_bf, out_cmp, glog)
    out = _out_proj(o_bf, _bf(Wo))
    return out[None]
